# baseline (speedup 1.0000x reference)
import functools

import jax
import jax.numpy as jnp
from jax import lax
from jax.experimental import pallas as pl
from jax.experimental.pallas import tpu as pltpu
from jax.experimental.pallas import tpu_sc as plsc

N = 10000
E = 320000
HID = 128
NUM_HIDDEN = 3

_NC = 2
_NS = 16
_NW = _NC * _NS
_CH = 128
_CPT = 80
_EPAD = _NW * _CPT * _CH
_NPAD = 10240
_RPS = _NPAD // _NS

_NBUF = 6
_NGRP = _CPT


def _sc_agg_body(h_hbm, src_hbm, dst_hbm, zeros_hbm, out_hbm,
                 src_v, dst_v, rows_v, gsem):
    cid = lax.axis_index("c")
    sid = lax.axis_index("s")
    wid = sid * _NC + cid

    pltpu.sync_copy(src_hbm.at[wid], src_v)
    pltpu.sync_copy(dst_hbm.at[wid], dst_v)
    plsc.subcore_barrier()

    def _gather(k, b):
        pltpu.async_copy(h_hbm.at[src_v.at[k]], rows_v.at[b], gsem.at[b])

    def _gwait(b):
        pltpu.make_async_copy(h_hbm.at[src_v.at[0]], rows_v.at[b],
                              gsem.at[b]).wait()

    for b in range(_NBUF):
        _gather(b, b)

    nfull = (_CPT // _NBUF) - 1

    def body(g, carry):
        k0 = g * _NBUF
        for b in range(_NBUF):
            _gwait(b)

            @pl.when(g < nfull)
            def _():
                _gather(k0 + _NBUF + b, b)
        return carry

    lax.fori_loop(0, _CPT // _NBUF, body, 0)
    for k in range(_CPT - (_CPT % _NBUF), _CPT):
        _gather(k, 0)
        _gwait(0)
    plsc.subcore_barrier()

    stripe = pl.multiple_of(sid * _RPS, 8)
    pltpu.sync_copy(zeros_hbm, out_hbm.at[cid, pl.ds(stripe, _RPS)])


@jax.jit
def _sc_agg(h, src, dst, zeros):
    mesh = plsc.VectorSubcoreMesh(core_axis_name="c", subcore_axis_name="s")
    return pl.kernel(
        _sc_agg_body,
        out_type=jax.ShapeDtypeStruct((_NC, _NPAD, HID), jnp.float32),
        mesh=mesh,
        scratch_types=[
            pltpu.VMEM((_CPT, _CH), jnp.int32),
            pltpu.VMEM((_CPT, _CH), jnp.int32),
            pltpu.VMEM((_NBUF, _CH, HID), jnp.float32),
            pltpu.SemaphoreType.DMA((_NBUF,)),
        ],
    )(h, src, dst, zeros)


_BLK = 1000


def _dense_block(h_ref, w_ref, b_ref, o_ref, *, relu):
    acc = jnp.dot(h_ref[...], w_ref[...],
                  preferred_element_type=jnp.float32) + b_ref[...]
    o_ref[...] = jnp.maximum(acc, 0.0) if relu else acc


def _dense(h, W, b2d, relu):
    dout = W.shape[1]
    return pl.pallas_call(
        functools.partial(_dense_block, relu=relu),
        grid=(N // _BLK,),
        in_specs=[
            pl.BlockSpec((_BLK, HID), lambda i: (i, 0)),
            pl.BlockSpec((HID, dout), lambda i: (0, 0)),
            pl.BlockSpec((1, dout), lambda i: (0, 0)),
        ],
        out_specs=pl.BlockSpec((_BLK, dout), lambda i: (i, 0)),
        out_shape=jax.ShapeDtypeStruct((N, dout), jnp.float32),
    )(h, W, b2d)


def _layer_block(h_ref, p_ref, w_ref, b_ref, o_ref):
    s = h_ref[...] + p_ref[0] + p_ref[1]
    acc = jnp.dot(s, w_ref[...], preferred_element_type=jnp.float32) + b_ref[...]
    o_ref[...] = jnp.maximum(acc, 0.0)


def _layer(h, parts, W, b2d):
    return pl.pallas_call(
        _layer_block,
        grid=(N // _BLK,),
        in_specs=[
            pl.BlockSpec((_BLK, HID), lambda i: (i, 0)),
            pl.BlockSpec((_NC, _BLK, HID), lambda i: (0, i, 0)),
            pl.BlockSpec((HID, HID), lambda i: (0, 0)),
            pl.BlockSpec((1, HID), lambda i: (0, 0)),
        ],
        out_specs=pl.BlockSpec((_BLK, HID), lambda i: (i, 0)),
        out_shape=jax.ShapeDtypeStruct((N, HID), jnp.float32),
    )(h, parts, W, b2d)


def kernel(x, edge_index, W_enc, b_enc, W_layers, b_layers, W_dec, b_dec):
    pad = _EPAD - E
    src = jnp.concatenate(
        [edge_index[0], jnp.zeros((pad,), jnp.int32)]).reshape(_NW, _CPT, _CH)
    dst = jnp.concatenate(
        [edge_index[1], jnp.full((pad,), N, jnp.int32)]).reshape(_NW, _CPT, _CH)
    zeros = jnp.zeros((_RPS, HID), jnp.float32)

    h = _dense(x, W_enc, b_enc.reshape(1, HID), relu=False)
    for i in range(NUM_HIDDEN):
        parts = _sc_agg(h, src, dst, zeros)
        h = _layer(h, parts, W_layers[i], b_layers[i].reshape(1, HID))

    Wd = jnp.pad(W_dec, ((0, 0), (0, HID - W_dec.shape[1])))
    bd = jnp.pad(b_dec, (0, HID - b_dec.shape[0])).reshape(1, HID)
    out = _dense(h, Wd, bd, relu=False)
    return out[:, : W_dec.shape[1]]


# speedup vs baseline: 2.2068x; 2.2068x over previous
import functools

import jax
import jax.numpy as jnp
from jax import lax
from jax.experimental import pallas as pl
from jax.experimental.pallas import tpu as pltpu
from jax.experimental.pallas import tpu_sc as plsc

N = 10000
E = 320000
HID = 128
NUM_HIDDEN = 3

_NC = 2
_NS = 16
_CH = 32
_CPT = 640
_EPAD = _NS * _CPT * _CH
_HALF = N // _NC
_HPAD = _HALF + 8
_ZROW = _HALF
_HRPS = 312
_ARPS = N // _NS
_ORPS = 624

_NBUF = 2
_IG = 4
_NGRP = _CPT // _IG


def _sc_agg_body(h_hbm, src_hbm, dst_hbm, zeros_hbm, out_hbm,
                 src_v, dst_v, rows_v, h_sh, acc_sh, gsem, isem):
    cid = lax.axis_index("c")
    sid = lax.axis_index("s")

    pltpu.sync_copy(h_hbm.at[pl.ds(cid * _HALF + sid * _HRPS, _HRPS)],
                    h_sh.at[pl.ds(sid * _HRPS, _HRPS)])

    @pl.when(sid == 0)
    def _():
        tail = _HALF - _NS * _HRPS
        pltpu.sync_copy(h_hbm.at[pl.ds(cid * _HALF + _NS * _HRPS, tail)],
                        h_sh.at[pl.ds(_NS * _HRPS, tail)])
        pltpu.sync_copy(zeros_hbm.at[pl.ds(0, _HPAD - _HALF)],
                        h_sh.at[pl.ds(_HALF, _HPAD - _HALF)])

    pltpu.sync_copy(zeros_hbm, acc_sh.at[pl.ds(sid * _ARPS, _ARPS)])

    def _iload(g, p, sem_slot):
        pltpu.async_copy(src_hbm.at[cid, sid, pl.ds(g * _IG, _IG)],
                         src_v.at[p], isem.at[sem_slot])
        pltpu.async_copy(dst_hbm.at[sid, pl.ds(g * _IG, _IG)],
                         dst_v.at[p], isem.at[sem_slot])

    def _iwait(sem_slot):
        for ref, buf in ((src_hbm.at[cid, sid, pl.ds(0, _IG)], src_v.at[0]),
                         (dst_hbm.at[sid, pl.ds(0, _IG)], dst_v.at[0])):
            pltpu.make_async_copy(ref, buf, isem.at[sem_slot]).wait()

    _iload(0, 0, 0)
    _iload(1, 1, 1)
    _iwait(0)
    plsc.subcore_barrier()

    def _gather(p, j, b):
        pltpu.async_copy(h_sh.at[src_v.at[p, j]], rows_v.at[b], gsem.at[b])

    def _scatter(p, j, b):
        pltpu.sync_copy(rows_v.at[b], acc_sh.at[dst_v.at[p, j]], add=True)

    def _gwait(b):
        pltpu.make_async_copy(h_sh.at[src_v.at[0, 0]], rows_v.at[b],
                              gsem.at[b]).wait()

    _gather(0, 0, 0)

    def body(g, carry):
        p = lax.rem(g, 2)
        for j in range(_IG):
            b = j % _NBUF
            _gwait(b)
            if j < _IG - 1:
                _gather(p, j + 1, (j + 1) % _NBUF)
            else:
                @pl.when(g < _NGRP - 1)
                def _():
                    _iwait(1 - p)
                    _gather(1 - p, 0, 0)
            _scatter(p, j, b)
        @pl.when(g < _NGRP - 2)
        def _():
            _iload(g + 2, p, p)
        return carry

    lax.fori_loop(0, _NGRP, body, 0)
    plsc.subcore_barrier()

    pltpu.sync_copy(acc_sh.at[pl.ds(sid * _ORPS, _ORPS)],
                    out_hbm.at[cid, pl.ds(sid * _ORPS, _ORPS)])

    @pl.when(sid == 0)
    def _():
        tail = N - _NS * _ORPS
        pltpu.sync_copy(acc_sh.at[pl.ds(_NS * _ORPS, tail)],
                        out_hbm.at[cid, pl.ds(_NS * _ORPS, tail)])


@jax.jit
def _sc_agg(h, src, dst, zeros):
    mesh = plsc.VectorSubcoreMesh(core_axis_name="c", subcore_axis_name="s")
    return pl.kernel(
        _sc_agg_body,
        out_type=jax.ShapeDtypeStruct((_NC, N, HID), jnp.float32),
        mesh=mesh,
        scratch_types=[
            pltpu.VMEM((2, _IG, _CH), jnp.int32),
            pltpu.VMEM((2, _IG, _CH), jnp.int32),
            pltpu.VMEM((_NBUF, _CH, HID), jnp.float32),
            pltpu.VMEM_SHARED((_HPAD, HID), jnp.float32),
            pltpu.VMEM_SHARED((N, HID), jnp.float32),
            pltpu.SemaphoreType.DMA((_NBUF,)),
            pltpu.SemaphoreType.DMA((2,)),
        ],
    )(h, src, dst, zeros)


_BLK = 1000


def _dense_block(h_ref, w_ref, b_ref, o_ref, *, relu):
    acc = jnp.dot(h_ref[...], w_ref[...],
                  preferred_element_type=jnp.float32) + b_ref[...]
    o_ref[...] = jnp.maximum(acc, 0.0) if relu else acc


def _dense(h, W, b2d, relu):
    dout = W.shape[1]
    return pl.pallas_call(
        functools.partial(_dense_block, relu=relu),
        grid=(N // _BLK,),
        in_specs=[
            pl.BlockSpec((_BLK, HID), lambda i: (i, 0)),
            pl.BlockSpec((HID, dout), lambda i: (0, 0)),
            pl.BlockSpec((1, dout), lambda i: (0, 0)),
        ],
        out_specs=pl.BlockSpec((_BLK, dout), lambda i: (i, 0)),
        out_shape=jax.ShapeDtypeStruct((N, dout), jnp.float32),
    )(h, W, b2d)


def _layer_block(h_ref, a0_ref, a1_ref, w_ref, b_ref, o_ref):
    s = h_ref[...] + a0_ref[0] + a1_ref[0]
    acc = jnp.dot(s, w_ref[...], preferred_element_type=jnp.float32) + b_ref[...]
    o_ref[...] = jnp.maximum(acc, 0.0)


def _layer(h, parts, W, b2d):
    return pl.pallas_call(
        _layer_block,
        grid=(N // _BLK,),
        in_specs=[
            pl.BlockSpec((_BLK, HID), lambda i: (i, 0)),
            pl.BlockSpec((1, _BLK, HID), lambda i: (0, i, 0)),
            pl.BlockSpec((1, _BLK, HID), lambda i: (1, i, 0)),
            pl.BlockSpec((HID, HID), lambda i: (0, 0)),
            pl.BlockSpec((1, HID), lambda i: (0, 0)),
        ],
        out_specs=pl.BlockSpec((_BLK, HID), lambda i: (i, 0)),
        out_shape=jax.ShapeDtypeStruct((N, HID), jnp.float32),
    )(h, parts, parts, W, b2d)


def kernel(x, edge_index, W_enc, b_enc, W_layers, b_layers, W_dec, b_dec):
    pad = _EPAD - E
    srcf = jnp.concatenate(
        [edge_index[0], jnp.full((pad,), -1, jnp.int32)])
    src = jnp.stack(
        [jnp.where((srcf >= c * _HALF) & (srcf < (c + 1) * _HALF),
                   srcf - c * _HALF, _ZROW)
         for c in range(_NC)]).reshape(_NC, _NS, _CPT, _CH)
    dst = jnp.concatenate(
        [edge_index[1], jnp.zeros((pad,), jnp.int32)]).reshape(_NS, _CPT, _CH)
    zeros = jnp.zeros((_ARPS, HID), jnp.float32)

    h = _dense(x, W_enc, b_enc.reshape(1, HID), relu=False)
    for i in range(NUM_HIDDEN):
        parts = _sc_agg(h, src, dst, zeros)
        h = _layer(h, parts, W_layers[i], b_layers[i].reshape(1, HID))

    Wd = jnp.pad(W_dec, ((0, 0), (0, HID - W_dec.shape[1])))
    bd = jnp.pad(b_dec, (0, HID - b_dec.shape[0])).reshape(1, HID)
    out = _dense(h, Wd, bd, relu=False)
    return out[:, : W_dec.shape[1]]
